# baseline (speedup 1.0000x reference)
import functools
import jax
import jax.numpy as jnp
from jax import lax
from jax.experimental import pallas as pl
from jax.experimental.pallas import tpu as pltpu
from jax.experimental.pallas import tpu_sc as plsc

B = 16384
D = 16
NH = 1000000
NV = 100000
NC = 2
NS = 16
L = 16
NW = NC * NS
BPW = B // NW
NCHUNK = 4
CHUNK = BPW // NCHUNK
NBLK = BPW // L
RPP = 128 // D

CW = 1920
HC_FULL = NH // CW
HC_REM_R0 = HC_FULL * CW
HC_REM_W = (NH - HC_REM_R0) // 128 * 128
HC_TAIL_R0 = HC_REM_R0 + HC_REM_W
HC_TAIL_W = NH - HC_TAIL_R0
VC_FULL = NV // CW
VC_REM_R0 = VC_FULL * CW
VC_REM_W = (NV - VC_REM_R0) // 128 * 128
VC_TAIL_R0 = VC_REM_R0 + VC_REM_W
VC_TAIL_W = NV - VC_TAIL_R0
FC_FULL = B // CW
FC_REM_R0 = FC_FULL * CW
FC_REM_W = B - FC_REM_R0
VPACK = NV // RPP + 4


def _sc_pack(htabT, vtabT, hfT, h_tail, v_tail):
    mesh = plsc.VectorSubcoreMesh(core_axis_name="c", subcore_axis_name="s")

    @functools.partial(
        pl.kernel,
        out_type=(
            jax.ShapeDtypeStruct((NH // RPP, 128), jnp.float32),
            jax.ShapeDtypeStruct((VPACK, 128), jnp.float32),
            jax.ShapeDtypeStruct((B // RPP, 128), jnp.float32),
        ),
        mesh=mesh,
        compiler_params=pltpu.CompilerParams(needs_layout_passes=False),
        scratch_types=[
            pltpu.VMEM((2, D, CW), jnp.float32),
            pltpu.VMEM((2, CW // RPP, 128), jnp.float32),
            pltpu.VMEM((D, 17), jnp.float32),
            pltpu.SemaphoreType.DMA,
            pltpu.SemaphoreType.DMA,
        ],
    )
    def pack_kernel(htab_hbm, vtab_hbm, hf_hbm, htail_hbm, vtail_hbm,
                    hout_hbm, vout_hbm, fout_hbm, blk2_v, out2_v, t16_v,
                    ssem, osem):
        wid = lax.axis_index("s") * NC + lax.axis_index("c")
        iota = lax.iota(jnp.int32, L)

        def transpose_block(blk_v, out_v, qq):
            def sb_body(k, carry):
                r0 = k * L
                for d in range(D):
                    t16_v[d, pl.ds(0, L)] = blk_v[d, pl.ds(r0, L)]
                for j in range(L):
                    col = plsc.load_gather(
                        t16_v, [iota, jnp.full((L,), j, jnp.int32)])
                    out_v[2 * k + j // RPP, pl.ds((j % RPP) * L, L)] = col
                return carry

            lax.fori_loop(0, qq // 2, sb_body, 0)

        def stage(src_hbm, r0, width, b):
            r0 = pl.multiple_of(r0, 128)
            return pltpu.async_copy(src_hbm.at[:, pl.ds(r0, width)],
                                    blk2_v.at[b, :, pl.ds(0, width)], ssem)

        def unstage(dst_hbm, r0, qq, b):
            row0 = pl.multiple_of(r0 // RPP, 8)
            return pltpu.async_copy(out2_v.at[b, pl.ds(0, qq)],
                                    dst_hbm.at[pl.ds(row0, qq)], osem)

        def do_chunk_sync(src_hbm, dst_hbm, r0, width, qq):
            stage(src_hbm, r0, width, 0).wait()
            transpose_block(blk2_v.at[0], out2_v.at[0], qq)
            unstage(dst_hbm, r0, qq, 0).wait()

        def pipe(src_hbm, dst_hbm, n, kmax):
            qq = CW // RPP
            for b in range(2):
                @pl.when(b < n)
                def _(_b=b):
                    stage(src_hbm, (_b * NW + wid) * CW, CW, _b)

            def k_body(k2, carry):
                for b in range(2):
                    t = 2 * k2 + b

                    @pl.when(t < n)
                    def _(_b=b, _t=t):
                        ci = _t * NW + wid
                        pltpu.make_async_copy(
                            htab_hbm.at[:, pl.ds(0, CW)],
                            blk2_v.at[_b, :, pl.ds(0, CW)], ssem).wait()

                        @pl.when(_t >= 2)
                        def _():
                            pltpu.make_async_copy(
                                hout_hbm.at[pl.ds(0, qq)],
                                out2_v.at[_b], osem).wait()

                        transpose_block(blk2_v.at[_b], out2_v.at[_b], qq)
                        unstage(dst_hbm, ci * CW, qq, _b)

                        @pl.when(_t + 2 < n)
                        def _():
                            stage(src_hbm, ((_t + 2) * NW + wid) * CW, CW, _b)
                return carry

            lax.fori_loop(0, kmax, k_body, 0)
            for b in range(2):
                @pl.when(b < n)
                def _(_b=b):
                    pltpu.make_async_copy(
                        hout_hbm.at[pl.ds(0, qq)],
                        out2_v.at[_b], osem).wait()

        nh_k = HC_FULL // NW + (wid < HC_FULL % NW).astype(jnp.int32)
        pipe(htab_hbm, hout_hbm, nh_k, (HC_FULL // NW + 2) // 2)

        nv_k = VC_FULL // NW + (wid < VC_FULL % NW).astype(jnp.int32)
        pipe(vtab_hbm, vout_hbm, nv_k, (VC_FULL // NW + 2) // 2)

        @pl.when(wid < FC_FULL)
        def _():
            do_chunk_sync(hf_hbm, fout_hbm, wid * CW, CW, CW // RPP)

        @pl.when(wid == 27)
        def _():
            if FC_REM_W:
                do_chunk_sync(hf_hbm, fout_hbm, FC_REM_R0, FC_REM_W,
                              FC_REM_W // RPP)

        @pl.when(wid == 31)
        def _():
            do_chunk_sync(htab_hbm, hout_hbm, HC_REM_R0, HC_REM_W, HC_REM_W // RPP)

        @pl.when(wid == 29)
        def _():
            do_chunk_sync(vtab_hbm, vout_hbm, VC_REM_R0, VC_REM_W, VC_REM_W // RPP)

        @pl.when(wid == 30)
        def _():
            pltpu.sync_copy(htail_hbm, out2_v.at[0, pl.ds(0, 8)])
            pltpu.sync_copy(out2_v.at[0, pl.ds(0, 8)],
                            hout_hbm.at[pl.ds(HC_TAIL_R0 // RPP, 8)])

        @pl.when(wid == 28)
        def _():
            pltpu.sync_copy(vtail_hbm, out2_v.at[0, pl.ds(0, 8)])
            pltpu.sync_copy(out2_v.at[0, pl.ds(0, 8)],
                            vout_hbm.at[pl.ds(VC_TAIL_R0 // RPP, 8)])

    return pack_kernel(htabT, vtabT, hfT, h_tail, v_tail)


def _sc_partials(h_idxs, v_idxs, hf2, htab2, vtab2):
    mesh = plsc.VectorSubcoreMesh(core_axis_name="c", subcore_axis_name="s")

    @functools.partial(
        pl.kernel,
        out_type=jax.ShapeDtypeStruct((NW, 128), jnp.float32),
        mesh=mesh,
        compiler_params=pltpu.CompilerParams(needs_layout_passes=False),
        scratch_types=[
            pltpu.VMEM((NCHUNK, CHUNK), jnp.int32),
            pltpu.VMEM((NCHUNK, CHUNK), jnp.int32),
            pltpu.VMEM((NCHUNK, CHUNK), jnp.int32),
            pltpu.VMEM((NCHUNK, CHUNK), jnp.int32),
            pltpu.VMEM((NBLK, L), jnp.int32),
            pltpu.VMEM((NBLK, L), jnp.int32),
            pltpu.VMEM((BPW, 128), jnp.float32),
            pltpu.VMEM((2, CHUNK, 128), jnp.float32),
            pltpu.VMEM((BPW // RPP, 128), jnp.float32),
            pltpu.VMEM((NBLK, L), jnp.float32),
            pltpu.VMEM((L, L), jnp.float32),
            pltpu.VMEM((128,), jnp.float32),
            pltpu.SemaphoreType.DMA,
            pltpu.SemaphoreType.DMA,
            pltpu.SemaphoreType.DMA,
        ],
    )
    def sc_kernel(hidx_hbm, vidx_hbm, hf_hbm, htab_hbm, vtab_hbm, out_hbm,
                  hraw_v, vraw_v, hg_v, vg_v, hoff_v, voff_v,
                  hrows_v, vring_v, hf_v, s_v, acc_v, t_v,
                  gsem, vsem, lsem):
        wid = lax.axis_index("s") * NC + lax.axis_index("c")

        pltpu.sync_copy(hidx_hbm.at[wid], hraw_v)
        pltpu.sync_copy(vidx_hbm.at[wid], vraw_v)

        for c in range(NCHUNK):
            for k in range(CHUNK // L):
                j = c * (CHUNK // L) + k
                hx = hraw_v[c, pl.ds(k * L, L)]
                vx = vraw_v[c, pl.ds(k * L, L)]
                hg_v[c, pl.ds(k * L, L)] = lax.shift_right_logical(hx, 3)
                vg_v[c, pl.ds(k * L, L)] = lax.shift_right_logical(vx, 3)
                hoff_v[j] = lax.shift_left(lax.bitwise_and(hx, 7), 4)
                voff_v[j] = lax.shift_left(lax.bitwise_and(vx, 7), 4)

        hf_cp = pltpu.async_copy(
            hf_hbm.at[pl.ds(wid * (BPW // RPP), BPW // RPP)], hf_v, lsem)
        h_cps = [
            pltpu.async_copy(htab_hbm.at[hg_v.at[c]],
                             hrows_v.at[pl.ds(c * CHUNK, CHUNK)], gsem)
            for c in range(NCHUNK)
        ]
        v_cps = [None] * NCHUNK
        for c in range(2):
            v_cps[c] = pltpu.async_copy(
                vtab_hbm.at[vg_v.at[c]], vring_v.at[c % 2], vsem)

        hf_cp.wait()
        for cp in h_cps:
            cp.wait()

        iota = lax.iota(jnp.int32, L)
        idiv = lax.shift_right_logical(iota, 3)
        colbase = lax.shift_left(lax.bitwise_and(iota, 7), 4)
        hf_cols = [colbase + e for e in range(D)]
        zero = jnp.zeros((L,), jnp.float32)

        def s_body(j, carry):
            rowv = j * L + iota
            rowhf = 2 * j + idiv
            hoffs = hoff_v[j]
            s = zero
            for e in range(D):
                h = plsc.load_gather(hrows_v, [rowv, hoffs + e])
                hf = plsc.load_gather(hf_v, [rowhf, hf_cols[e]])
                s = s + h * hf
            s_v[j] = s
            return carry

        lax.fori_loop(0, NBLK, s_body, 0)

        blk_per_chunk = CHUNK // L
        accs = tuple(zero for _ in range(D))
        for c in range(NCHUNK):
            v_cps[c].wait()
            if c + 2 < NCHUNK:
                v_cps[c + 2] = pltpu.async_copy(
                    vtab_hbm.at[vg_v.at[c + 2]], vring_v.at[c % 2], vsem)
            vbuf = vring_v.at[c % 2]

            def v_body(k, accs, _c=c, _vbuf=vbuf):
                j = _c * blk_per_chunk + k
                rowv = k * L + iota
                voffs = voff_v[j]
                s = s_v[j]
                return tuple(
                    accs[d] + s * plsc.load_gather(_vbuf, [rowv, voffs + d])
                    for d in range(D))

            accs = lax.fori_loop(0, blk_per_chunk, v_body, accs)

        for d in range(D):
            acc_v[d] = accs[d]
        t = zero
        cols = [jnp.full((L,), i, jnp.int32) for i in range(L)]
        for i in range(L):
            t = t + plsc.load_gather(acc_v, [iota, cols[i]])
        for r in range(RPP):
            t_v[pl.ds(r * L, L)] = t
        pltpu.sync_copy(t_v, out_hbm.at[wid])

    return sc_kernel(h_idxs, v_idxs, hf2, htab2, vtab2)


def _tc_finish(partials, v_feats_t):
    def tc_kernel(p_ref, vft_ref, o_ref):
        t_rep = jnp.sum(p_ref[...], axis=0)
        t = t_rep[:D].reshape(1, D)
        o_ref[...] = jnp.dot(t, vft_ref[...],
                             preferred_element_type=jnp.float32)

    return pl.pallas_call(
        tc_kernel,
        out_shape=jax.ShapeDtypeStruct((1, B), jnp.float32),
    )(partials, v_feats_t)


def kernel(h_idxs, v_idxs, h_feats, v_feats, human_table, virus_table):
    h_idxs = h_idxs.astype(jnp.int32).reshape(NW, NCHUNK, CHUNK)
    v_idxs = v_idxs.astype(jnp.int32).reshape(NW, NCHUNK, CHUNK)
    h_tail = human_table[HC_TAIL_R0:].reshape(8, 128)
    v_tail = jnp.pad(virus_table[VC_TAIL_R0:], ((0, 32), (0, 0))).reshape(8, 128)
    htab2, vtab2, hf2 = _sc_pack(human_table.T, virus_table.T, h_feats.T,
                                 h_tail, v_tail)
    partials = _sc_partials(h_idxs, v_idxs, hf2, htab2, vtab2)
    out = _tc_finish(partials, v_feats.T)
    return out.reshape(B)


# speedup vs baseline: 3.0727x; 3.0727x over previous
import functools
import jax
import jax.numpy as jnp
from jax import lax
from jax.experimental import pallas as pl
from jax.experimental.pallas import tpu as pltpu
from jax.experimental.pallas import tpu_sc as plsc

B = 16384
D = 16
NH = 1000000
NV = 100000
NC = 2
NS = 16
L = 16
NW = NC * NS
BPW = B // NW
NCHUNK = 4
CHUNK = BPW // NCHUNK
NBLK = BPW // L
RPP = 128 // D

CW = 1920
HC_FULL = NH // CW
HC_REM_R0 = HC_FULL * CW
HC_REM_W = (NH - HC_REM_R0) // 128 * 128
HC_TAIL_R0 = HC_REM_R0 + HC_REM_W
HC_TAIL_W = NH - HC_TAIL_R0
VC_FULL = NV // CW
VC_REM_R0 = VC_FULL * CW
VC_REM_W = (NV - VC_REM_R0) // 128 * 128
VC_TAIL_R0 = VC_REM_R0 + VC_REM_W
VC_TAIL_W = NV - VC_TAIL_R0
FC_FULL = B // CW
FC_REM_R0 = FC_FULL * CW
FC_REM_W = B - FC_REM_R0
VPACK = NV // RPP + 4


def _sc_pack(htabT, vtabT, hfT, h_tail, v_tail):
    mesh = plsc.VectorSubcoreMesh(core_axis_name="c", subcore_axis_name="s")

    @functools.partial(
        pl.kernel,
        out_type=(
            jax.ShapeDtypeStruct((NH // RPP, 128), jnp.float32),
            jax.ShapeDtypeStruct((VPACK, 128), jnp.float32),
            jax.ShapeDtypeStruct((B // RPP, 128), jnp.float32),
        ),
        mesh=mesh,
        compiler_params=pltpu.CompilerParams(needs_layout_passes=False),
        scratch_types=[
            pltpu.VMEM((2, D, CW), jnp.float32),
            pltpu.VMEM((2, CW // RPP, 128), jnp.float32),
            pltpu.SemaphoreType.DMA,
            pltpu.SemaphoreType.DMA,
        ],
    )
    def pack_kernel(htab_hbm, vtab_hbm, hf_hbm, htail_hbm, vtail_hbm,
                    hout_hbm, vout_hbm, fout_hbm, blk2_v, out2_v, ssem, osem):
        wid = lax.axis_index("s") * NC + lax.axis_index("c")
        iota = lax.iota(jnp.int32, L)
        rl_j = [lax.bitwise_and(iota + j, 15) for j in range(L)]
        rowoff_j = [lax.shift_right_logical(r, 3) for r in rl_j]
        coloff_j = [lax.shift_left(lax.bitwise_and(r, 7), 4) + iota
                    for r in rl_j]

        def transpose_block(blk_v, out_v, qq):
            def sb_body(k, carry):
                r0 = k * L
                q0 = 2 * k
                for j in range(L):
                    vals = plsc.load_gather(blk_v, [iota, r0 + rl_j[j]])
                    plsc.store_scatter(out_v, [q0 + rowoff_j[j], coloff_j[j]],
                                       vals)
                return carry

            lax.fori_loop(0, qq // 2, sb_body, 0)

        def stage(src_hbm, r0, width, b):
            r0 = pl.multiple_of(r0, 128)
            return pltpu.async_copy(src_hbm.at[:, pl.ds(r0, width)],
                                    blk2_v.at[b, :, pl.ds(0, width)], ssem)

        def unstage(dst_hbm, r0, qq, b):
            row0 = pl.multiple_of(r0 // RPP, 8)
            return pltpu.async_copy(out2_v.at[b, pl.ds(0, qq)],
                                    dst_hbm.at[pl.ds(row0, qq)], osem)

        def do_chunk_sync(src_hbm, dst_hbm, r0, width, qq):
            stage(src_hbm, r0, width, 0).wait()
            transpose_block(blk2_v.at[0], out2_v.at[0], qq)
            unstage(dst_hbm, r0, qq, 0).wait()

        def pipe(src_hbm, dst_hbm, n, kmax):
            qq = CW // RPP
            for b in range(2):
                @pl.when(b < n)
                def _(_b=b):
                    stage(src_hbm, (_b * NW + wid) * CW, CW, _b)

            def k_body(k2, carry):
                for b in range(2):
                    t = 2 * k2 + b

                    @pl.when(t < n)
                    def _(_b=b, _t=t):
                        ci = _t * NW + wid
                        pltpu.make_async_copy(
                            htab_hbm.at[:, pl.ds(0, CW)],
                            blk2_v.at[_b, :, pl.ds(0, CW)], ssem).wait()

                        @pl.when(_t >= 2)
                        def _():
                            pltpu.make_async_copy(
                                hout_hbm.at[pl.ds(0, qq)],
                                out2_v.at[_b], osem).wait()

                        transpose_block(blk2_v.at[_b], out2_v.at[_b], qq)
                        unstage(dst_hbm, ci * CW, qq, _b)

                        @pl.when(_t + 2 < n)
                        def _():
                            stage(src_hbm, ((_t + 2) * NW + wid) * CW, CW, _b)
                return carry

            lax.fori_loop(0, kmax, k_body, 0)
            for b in range(2):
                @pl.when(b < n)
                def _(_b=b):
                    pltpu.make_async_copy(
                        hout_hbm.at[pl.ds(0, qq)],
                        out2_v.at[_b], osem).wait()

        nh_k = HC_FULL // NW + (wid < HC_FULL % NW).astype(jnp.int32)
        pipe(htab_hbm, hout_hbm, nh_k, (HC_FULL // NW + 2) // 2)

        nv_k = VC_FULL // NW + (wid < VC_FULL % NW).astype(jnp.int32)
        pipe(vtab_hbm, vout_hbm, nv_k, (VC_FULL // NW + 2) // 2)

        @pl.when(wid < FC_FULL)
        def _():
            do_chunk_sync(hf_hbm, fout_hbm, wid * CW, CW, CW // RPP)

        @pl.when(wid == 27)
        def _():
            if FC_REM_W:
                do_chunk_sync(hf_hbm, fout_hbm, FC_REM_R0, FC_REM_W,
                              FC_REM_W // RPP)

        @pl.when(wid == 31)
        def _():
            do_chunk_sync(htab_hbm, hout_hbm, HC_REM_R0, HC_REM_W,
                          HC_REM_W // RPP)

        @pl.when(wid == 29)
        def _():
            do_chunk_sync(vtab_hbm, vout_hbm, VC_REM_R0, VC_REM_W,
                          VC_REM_W // RPP)

        @pl.when(wid == 30)
        def _():
            pltpu.sync_copy(htail_hbm, out2_v.at[0, pl.ds(0, 8)])
            pltpu.sync_copy(out2_v.at[0, pl.ds(0, 8)],
                            hout_hbm.at[pl.ds(HC_TAIL_R0 // RPP, 8)])

        @pl.when(wid == 28)
        def _():
            pltpu.sync_copy(vtail_hbm, out2_v.at[0, pl.ds(0, 8)])
            pltpu.sync_copy(out2_v.at[0, pl.ds(0, 8)],
                            vout_hbm.at[pl.ds(VC_TAIL_R0 // RPP, 8)])

    return pack_kernel(htabT, vtabT, hfT, h_tail, v_tail)


def _sc_partials(h_idxs, v_idxs, hf2, htab2, vtab2):
    mesh = plsc.VectorSubcoreMesh(core_axis_name="c", subcore_axis_name="s")

    @functools.partial(
        pl.kernel,
        out_type=jax.ShapeDtypeStruct((NW, 128), jnp.float32),
        mesh=mesh,
        compiler_params=pltpu.CompilerParams(needs_layout_passes=False),
        scratch_types=[
            pltpu.VMEM((NCHUNK, CHUNK), jnp.int32),
            pltpu.VMEM((NCHUNK, CHUNK), jnp.int32),
            pltpu.VMEM((NCHUNK, CHUNK), jnp.int32),
            pltpu.VMEM((NCHUNK, CHUNK), jnp.int32),
            pltpu.VMEM((NBLK, L), jnp.int32),
            pltpu.VMEM((NBLK, L), jnp.int32),
            pltpu.VMEM((BPW, 128), jnp.float32),
            pltpu.VMEM((2, CHUNK, 128), jnp.float32),
            pltpu.VMEM((BPW // RPP, 128), jnp.float32),
            pltpu.VMEM((NBLK, L), jnp.float32),
            pltpu.VMEM((L, L), jnp.float32),
            pltpu.VMEM((128,), jnp.float32),
            pltpu.SemaphoreType.DMA,
            pltpu.SemaphoreType.DMA,
            pltpu.SemaphoreType.DMA,
        ],
    )
    def sc_kernel(hidx_hbm, vidx_hbm, hf_hbm, htab_hbm, vtab_hbm, out_hbm,
                  hraw_v, vraw_v, hg_v, vg_v, hoff_v, voff_v,
                  hrows_v, vring_v, hf_v, s_v, acc_v, t_v,
                  gsem, vsem, lsem):
        wid = lax.axis_index("s") * NC + lax.axis_index("c")

        pltpu.sync_copy(hidx_hbm.at[wid], hraw_v)
        pltpu.sync_copy(vidx_hbm.at[wid], vraw_v)

        for c in range(NCHUNK):
            for k in range(CHUNK // L):
                j = c * (CHUNK // L) + k
                hx = hraw_v[c, pl.ds(k * L, L)]
                vx = vraw_v[c, pl.ds(k * L, L)]
                hg_v[c, pl.ds(k * L, L)] = lax.shift_right_logical(hx, 3)
                vg_v[c, pl.ds(k * L, L)] = lax.shift_right_logical(vx, 3)
                hoff_v[j] = lax.shift_left(lax.bitwise_and(hx, 7), 4)
                voff_v[j] = lax.shift_left(lax.bitwise_and(vx, 7), 4)

        hf_cp = pltpu.async_copy(
            hf_hbm.at[pl.ds(wid * (BPW // RPP), BPW // RPP)], hf_v, lsem)
        h_cps = [
            pltpu.async_copy(htab_hbm.at[hg_v.at[c]],
                             hrows_v.at[pl.ds(c * CHUNK, CHUNK)], gsem)
            for c in range(NCHUNK)
        ]
        v_cps = [None] * NCHUNK
        for c in range(2):
            v_cps[c] = pltpu.async_copy(
                vtab_hbm.at[vg_v.at[c]], vring_v.at[c % 2], vsem)

        hf_cp.wait()
        for cp in h_cps:
            cp.wait()

        iota = lax.iota(jnp.int32, L)
        idiv = lax.shift_right_logical(iota, 3)
        colbase = lax.shift_left(lax.bitwise_and(iota, 7), 4)
        hf_cols = [colbase + e for e in range(D)]
        zero = jnp.zeros((L,), jnp.float32)

        def s_body(j, carry):
            rowv = j * L + iota
            rowhf = 2 * j + idiv
            hoffs = hoff_v[j]
            s = zero
            for e in range(D):
                h = plsc.load_gather(hrows_v, [rowv, hoffs + e])
                hf = plsc.load_gather(hf_v, [rowhf, hf_cols[e]])
                s = s + h * hf
            s_v[j] = s
            return carry

        lax.fori_loop(0, NBLK, s_body, 0)

        blk_per_chunk = CHUNK // L
        accs = tuple(zero for _ in range(D))
        for c in range(NCHUNK):
            v_cps[c].wait()
            if c + 2 < NCHUNK:
                v_cps[c + 2] = pltpu.async_copy(
                    vtab_hbm.at[vg_v.at[c + 2]], vring_v.at[c % 2], vsem)
            vbuf = vring_v.at[c % 2]

            def v_body(k, accs, _c=c, _vbuf=vbuf):
                j = _c * blk_per_chunk + k
                rowv = k * L + iota
                voffs = voff_v[j]
                s = s_v[j]
                return tuple(
                    accs[d] + s * plsc.load_gather(_vbuf, [rowv, voffs + d])
                    for d in range(D))

            accs = lax.fori_loop(0, blk_per_chunk, v_body, accs)

        for d in range(D):
            acc_v[d] = accs[d]
        t = zero
        cols = [jnp.full((L,), i, jnp.int32) for i in range(L)]
        for i in range(L):
            t = t + plsc.load_gather(acc_v, [iota, cols[i]])
        for r in range(RPP):
            t_v[pl.ds(r * L, L)] = t
        pltpu.sync_copy(t_v, out_hbm.at[wid])

    return sc_kernel(h_idxs, v_idxs, hf2, htab2, vtab2)


def _tc_finish(partials, v_feats_t):
    def tc_kernel(p_ref, vft_ref, o_ref):
        t_rep = jnp.sum(p_ref[...], axis=0)
        t = t_rep[:D].reshape(1, D)
        o_ref[...] = jnp.dot(t, vft_ref[...],
                             preferred_element_type=jnp.float32)

    return pl.pallas_call(
        tc_kernel,
        out_shape=jax.ShapeDtypeStruct((1, B), jnp.float32),
    )(partials, v_feats_t)


def kernel(h_idxs, v_idxs, h_feats, v_feats, human_table, virus_table):
    h_idxs = h_idxs.astype(jnp.int32).reshape(NW, NCHUNK, CHUNK)
    v_idxs = v_idxs.astype(jnp.int32).reshape(NW, NCHUNK, CHUNK)
    h_tail = human_table[HC_TAIL_R0:].reshape(8, 128)
    v_tail = jnp.pad(virus_table[VC_TAIL_R0:], ((0, 32), (0, 0))).reshape(8, 128)
    htab2, vtab2, hf2 = _sc_pack(human_table.T, virus_table.T, h_feats.T,
                                 h_tail, v_tail)
    partials = _sc_partials(h_idxs, v_idxs, hf2, htab2, vtab2)
    out = _tc_finish(partials, v_feats.T)
    return out.reshape(B)


# speedup vs baseline: 3.0938x; 1.0069x over previous
import functools
import jax
import jax.numpy as jnp
from jax import lax
from jax.experimental import pallas as pl
from jax.experimental.pallas import tpu as pltpu
from jax.experimental.pallas import tpu_sc as plsc

B = 16384
D = 16
NH = 1000000
NV = 100000
NC = 2
NS = 16
L = 16
NW = NC * NS
BPW = B // NW
NCHUNK = 4
CHUNK = BPW // NCHUNK
NBLK = BPW // L
RPP = 128 // D

CW = 1920
HC_FULL = NH // CW
HC_REM_R0 = HC_FULL * CW
HC_REM_W = (NH - HC_REM_R0) // 128 * 128
HC_TAIL_R0 = HC_REM_R0 + HC_REM_W
HC_TAIL_W = NH - HC_TAIL_R0
VC_FULL = NV // CW
VC_REM_R0 = VC_FULL * CW
VC_REM_W = (NV - VC_REM_R0) // 128 * 128
VC_TAIL_R0 = VC_REM_R0 + VC_REM_W
VC_TAIL_W = NV - VC_TAIL_R0
FC_FULL = B // CW
FC_REM_R0 = FC_FULL * CW
FC_REM_W = B - FC_REM_R0
VPACK = NV // RPP + 4


def _sc_pack(htabT, vtabT, hfT, h_tail, v_tail):
    mesh = plsc.VectorSubcoreMesh(core_axis_name="c", subcore_axis_name="s")

    @functools.partial(
        pl.kernel,
        out_type=(
            jax.ShapeDtypeStruct((NH // RPP, 128), jnp.float32),
            jax.ShapeDtypeStruct((VPACK, 128), jnp.float32),
            jax.ShapeDtypeStruct((B // RPP, 128), jnp.float32),
        ),
        mesh=mesh,
        compiler_params=pltpu.CompilerParams(needs_layout_passes=False),
        scratch_types=[
            pltpu.VMEM((2, D, CW), jnp.float32),
            pltpu.VMEM((2, CW // RPP, 128), jnp.float32),
            pltpu.SemaphoreType.DMA,
            pltpu.SemaphoreType.DMA,
        ],
    )
    def pack_kernel(htab_hbm, vtab_hbm, hf_hbm, htail_hbm, vtail_hbm,
                    hout_hbm, vout_hbm, fout_hbm, blk2_v, out2_v, ssem, osem):
        wid = lax.axis_index("s") * NC + lax.axis_index("c")
        iota = lax.iota(jnp.int32, L)
        rl_j = [lax.bitwise_and(iota + j, 15) for j in range(L)]
        rowoff_j = [lax.shift_right_logical(r, 3) for r in rl_j]
        coloff_j = [lax.shift_left(lax.bitwise_and(r, 7), 4) + iota
                    for r in rl_j]

        def transpose_block(blk_v, out_v, qq):
            def sb_body(k, carry):
                for u in range(2):
                    r0 = (2 * k + u) * L
                    q0 = 2 * (2 * k + u)
                    for j in range(L):
                        vals = plsc.load_gather(blk_v, [iota, r0 + rl_j[j]])
                        plsc.store_scatter(
                            out_v, [q0 + rowoff_j[j], coloff_j[j]], vals)
                return carry

            lax.fori_loop(0, qq // 4, sb_body, 0)

        def stage(src_hbm, r0, width, b):
            r0 = pl.multiple_of(r0, 128)
            return pltpu.async_copy(src_hbm.at[:, pl.ds(r0, width)],
                                    blk2_v.at[b, :, pl.ds(0, width)], ssem)

        def unstage(dst_hbm, r0, qq, b):
            row0 = pl.multiple_of(r0 // RPP, 8)
            return pltpu.async_copy(out2_v.at[b, pl.ds(0, qq)],
                                    dst_hbm.at[pl.ds(row0, qq)], osem)

        def do_chunk_sync(src_hbm, dst_hbm, r0, width, qq):
            stage(src_hbm, r0, width, 0).wait()
            transpose_block(blk2_v.at[0], out2_v.at[0], qq)
            unstage(dst_hbm, r0, qq, 0).wait()

        def pipe(src_hbm, dst_hbm, n, kmax):
            qq = CW // RPP
            for b in range(2):
                @pl.when(b < n)
                def _(_b=b):
                    stage(src_hbm, (_b * NW + wid) * CW, CW, _b)

            def k_body(k2, carry):
                for b in range(2):
                    t = 2 * k2 + b

                    @pl.when(t < n)
                    def _(_b=b, _t=t):
                        ci = _t * NW + wid
                        pltpu.make_async_copy(
                            htab_hbm.at[:, pl.ds(0, CW)],
                            blk2_v.at[_b, :, pl.ds(0, CW)], ssem).wait()

                        @pl.when(_t >= 2)
                        def _():
                            pltpu.make_async_copy(
                                hout_hbm.at[pl.ds(0, qq)],
                                out2_v.at[_b], osem).wait()

                        transpose_block(blk2_v.at[_b], out2_v.at[_b], qq)
                        unstage(dst_hbm, ci * CW, qq, _b)

                        @pl.when(_t + 2 < n)
                        def _():
                            stage(src_hbm, ((_t + 2) * NW + wid) * CW, CW, _b)
                return carry

            lax.fori_loop(0, kmax, k_body, 0)
            for b in range(2):
                @pl.when(b < n)
                def _(_b=b):
                    pltpu.make_async_copy(
                        hout_hbm.at[pl.ds(0, qq)],
                        out2_v.at[_b], osem).wait()

        nh_k = HC_FULL // NW + (wid < HC_FULL % NW).astype(jnp.int32)
        pipe(htab_hbm, hout_hbm, nh_k, (HC_FULL // NW + 2) // 2)

        nv_k = VC_FULL // NW + (wid < VC_FULL % NW).astype(jnp.int32)
        pipe(vtab_hbm, vout_hbm, nv_k, (VC_FULL // NW + 2) // 2)

        @pl.when(wid < FC_FULL)
        def _():
            do_chunk_sync(hf_hbm, fout_hbm, wid * CW, CW, CW // RPP)

        @pl.when(wid == 27)
        def _():
            if FC_REM_W:
                do_chunk_sync(hf_hbm, fout_hbm, FC_REM_R0, FC_REM_W,
                              FC_REM_W // RPP)

        @pl.when(wid == 31)
        def _():
            do_chunk_sync(htab_hbm, hout_hbm, HC_REM_R0, HC_REM_W,
                          HC_REM_W // RPP)

        @pl.when(wid == 29)
        def _():
            do_chunk_sync(vtab_hbm, vout_hbm, VC_REM_R0, VC_REM_W,
                          VC_REM_W // RPP)

        @pl.when(wid == 30)
        def _():
            pltpu.sync_copy(htail_hbm, out2_v.at[0, pl.ds(0, 8)])
            pltpu.sync_copy(out2_v.at[0, pl.ds(0, 8)],
                            hout_hbm.at[pl.ds(HC_TAIL_R0 // RPP, 8)])

        @pl.when(wid == 28)
        def _():
            pltpu.sync_copy(vtail_hbm, out2_v.at[0, pl.ds(0, 8)])
            pltpu.sync_copy(out2_v.at[0, pl.ds(0, 8)],
                            vout_hbm.at[pl.ds(VC_TAIL_R0 // RPP, 8)])

    return pack_kernel(htabT, vtabT, hfT, h_tail, v_tail)


def _sc_partials(h_idxs, v_idxs, hf2, htab2, vtab2):
    mesh = plsc.VectorSubcoreMesh(core_axis_name="c", subcore_axis_name="s")

    @functools.partial(
        pl.kernel,
        out_type=jax.ShapeDtypeStruct((NW, 128), jnp.float32),
        mesh=mesh,
        compiler_params=pltpu.CompilerParams(needs_layout_passes=False),
        scratch_types=[
            pltpu.VMEM((NCHUNK, CHUNK), jnp.int32),
            pltpu.VMEM((NCHUNK, CHUNK), jnp.int32),
            pltpu.VMEM((NCHUNK, CHUNK), jnp.int32),
            pltpu.VMEM((NCHUNK, CHUNK), jnp.int32),
            pltpu.VMEM((NBLK, L), jnp.int32),
            pltpu.VMEM((NBLK, L), jnp.int32),
            pltpu.VMEM((BPW, 128), jnp.float32),
            pltpu.VMEM((2, CHUNK, 128), jnp.float32),
            pltpu.VMEM((BPW // RPP, 128), jnp.float32),
            pltpu.VMEM((NBLK, L), jnp.float32),
            pltpu.VMEM((L, L), jnp.float32),
            pltpu.VMEM((128,), jnp.float32),
            pltpu.SemaphoreType.DMA,
            pltpu.SemaphoreType.DMA,
            pltpu.SemaphoreType.DMA,
        ],
    )
    def sc_kernel(hidx_hbm, vidx_hbm, hf_hbm, htab_hbm, vtab_hbm, out_hbm,
                  hraw_v, vraw_v, hg_v, vg_v, hoff_v, voff_v,
                  hrows_v, vring_v, hf_v, s_v, acc_v, t_v,
                  gsem, vsem, lsem):
        wid = lax.axis_index("s") * NC + lax.axis_index("c")

        pltpu.sync_copy(hidx_hbm.at[wid], hraw_v)
        pltpu.sync_copy(vidx_hbm.at[wid], vraw_v)

        for c in range(NCHUNK):
            for k in range(CHUNK // L):
                j = c * (CHUNK // L) + k
                hx = hraw_v[c, pl.ds(k * L, L)]
                vx = vraw_v[c, pl.ds(k * L, L)]
                hg_v[c, pl.ds(k * L, L)] = lax.shift_right_logical(hx, 3)
                vg_v[c, pl.ds(k * L, L)] = lax.shift_right_logical(vx, 3)
                hoff_v[j] = lax.shift_left(lax.bitwise_and(hx, 7), 4)
                voff_v[j] = lax.shift_left(lax.bitwise_and(vx, 7), 4)

        hf_cp = pltpu.async_copy(
            hf_hbm.at[pl.ds(wid * (BPW // RPP), BPW // RPP)], hf_v, lsem)
        h_cps = [
            pltpu.async_copy(htab_hbm.at[hg_v.at[c]],
                             hrows_v.at[pl.ds(c * CHUNK, CHUNK)], gsem)
            for c in range(NCHUNK)
        ]
        v_cps = [None] * NCHUNK
        for c in range(2):
            v_cps[c] = pltpu.async_copy(
                vtab_hbm.at[vg_v.at[c]], vring_v.at[c % 2], vsem)

        hf_cp.wait()
        for cp in h_cps:
            cp.wait()

        iota = lax.iota(jnp.int32, L)
        idiv = lax.shift_right_logical(iota, 3)
        colbase = lax.shift_left(lax.bitwise_and(iota, 7), 4)
        diag_e = [lax.bitwise_and(iota + e, 15) for e in range(D)]
        hf_cols = [colbase + diag_e[e] for e in range(D)]
        zero = jnp.zeros((L,), jnp.float32)

        def s_body(j, carry):
            rowv = j * L + iota
            rowhf = 2 * j + idiv
            hoffs = hoff_v[j]
            s = zero
            for e in range(D):
                h = plsc.load_gather(hrows_v, [rowv, hoffs + diag_e[e]])
                hf = plsc.load_gather(hf_v, [rowhf, hf_cols[e]])
                s = s + h * hf
            s_v[j] = s
            return carry

        lax.fori_loop(0, NBLK, s_body, 0)

        blk_per_chunk = CHUNK // L
        accs = tuple(zero for _ in range(D))
        for c in range(NCHUNK):
            v_cps[c].wait()
            if c + 2 < NCHUNK:
                v_cps[c + 2] = pltpu.async_copy(
                    vtab_hbm.at[vg_v.at[c + 2]], vring_v.at[c % 2], vsem)
            vbuf = vring_v.at[c % 2]

            def v_body(k, accs, _c=c, _vbuf=vbuf):
                j = _c * blk_per_chunk + k
                rowv = k * L + iota
                voffs = voff_v[j]
                s = s_v[j]
                return tuple(
                    accs[e] + s * plsc.load_gather(
                        _vbuf, [rowv, voffs + diag_e[e]])
                    for e in range(D))

            accs = lax.fori_loop(0, blk_per_chunk, v_body, accs)

        for e in range(D):
            acc_v[e] = accs[e]
        t = zero
        for i in range(L):
            rows = lax.bitwise_and(iota - i, 15)
            t = t + plsc.load_gather(acc_v, [rows, jnp.full((L,), i,
                                                            jnp.int32)])
        for r in range(RPP):
            t_v[pl.ds(r * L, L)] = t
        pltpu.sync_copy(t_v, out_hbm.at[wid])

    return sc_kernel(h_idxs, v_idxs, hf2, htab2, vtab2)


def _tc_finish(partials, v_feats_t):
    def tc_kernel(p_ref, vft_ref, o_ref):
        t_rep = jnp.sum(p_ref[...], axis=0)
        t = t_rep[:D].reshape(1, D)
        o_ref[...] = jnp.dot(t, vft_ref[...],
                             preferred_element_type=jnp.float32)

    return pl.pallas_call(
        tc_kernel,
        out_shape=jax.ShapeDtypeStruct((1, B), jnp.float32),
    )(partials, v_feats_t)


def kernel(h_idxs, v_idxs, h_feats, v_feats, human_table, virus_table):
    h_idxs = h_idxs.astype(jnp.int32).reshape(NW, NCHUNK, CHUNK)
    v_idxs = v_idxs.astype(jnp.int32).reshape(NW, NCHUNK, CHUNK)
    h_tail = human_table[HC_TAIL_R0:].reshape(8, 128)
    v_tail = jnp.pad(virus_table[VC_TAIL_R0:], ((0, 32), (0, 0))).reshape(8, 128)
    htab2, vtab2, hf2 = _sc_pack(human_table.T, virus_table.T, h_feats.T,
                                 h_tail, v_tail)
    partials = _sc_partials(h_idxs, v_idxs, hf2, htab2, vtab2)
    out = _tc_finish(partials, v_feats.T)
    return out.reshape(B)
